# baseline (speedup 1.0000x reference)
import functools

import jax
import jax.numpy as jnp
from jax import lax
from jax.experimental import pallas as pl
from jax.experimental.pallas import tpu as pltpu
from jax.experimental.pallas import tpu_sc as plsc

N = 10000
E = 320000
D = 128
L = 4
EPS = 1e-5

NC = 2
NS = 16
NW = NC * NS

CHUNK = 64
CH_PER_TILE = 160
EDGES_PER_TILE = CH_PER_TILE * CHUNK
E_PAD = NW * EDGES_PER_TILE
DEG_CHUNK = 128
DEG_CPT = EDGES_PER_TILE // DEG_CHUNK
NPAD = 10240
ROWS_PER_TILE_PAD = NPAD // NS

_mesh = plsc.VectorSubcoreMesh(
    core_axis_name="c", subcore_axis_name="s", num_cores=NC, num_subcores=NS
)


@functools.partial(
    pl.kernel,
    out_type=jax.ShapeDtypeStruct((NC, NPAD), jnp.float32),
    mesh=_mesh,
    scratch_types=[
        pltpu.VMEM_SHARED((NPAD,), jnp.float32),
        pltpu.VMEM((DEG_CHUNK,), jnp.int32),
        pltpu.VMEM((DEG_CHUNK,), jnp.int32),
        pltpu.VMEM((DEG_CHUNK,), jnp.float32),
        pltpu.VMEM((ROWS_PER_TILE_PAD,), jnp.float32),
        pltpu.SemaphoreType.DMA,
        pltpu.SemaphoreType.DMA,
    ],
)
def _sc_deg(col_hbm, deg_hbm, deg_sh, ci0, ci1, ones_v, dbuf, si0, si1):
    cid = lax.axis_index("c")
    sid = lax.axis_index("s")
    w = cid * NS + sid

    z16 = jnp.zeros((16,), jnp.float32)
    o16 = jnp.ones((16,), jnp.float32)
    for i in range(DEG_CHUNK // 16):
        ones_v[pl.ds(i * 16, 16)] = o16

    def zbody(i, _):
        dbuf[pl.ds(i * 16, 16)] = z16
        return 0

    lax.fori_loop(0, ROWS_PER_TILE_PAD // 16, zbody, 0)
    pltpu.sync_copy(dbuf, deg_sh.at[pl.ds(sid * ROWS_PER_TILE_PAD, ROWS_PER_TILE_PAD)])
    plsc.subcore_barrier()

    def idxc(g, ci, si):
        base = pl.multiple_of(w * EDGES_PER_TILE + g * DEG_CHUNK, DEG_CHUNK)
        pltpu.async_copy(col_hbm.at[pl.ds(base, DEG_CHUNK)], ci, si)

    def widx(ci, si):
        pltpu.make_async_copy(col_hbm.at[pl.ds(0, DEG_CHUNK)], ci, si).wait()

    idxc(0, ci0, si0)
    idxc(1, ci1, si1)

    def body(k, _):
        g = k * 2
        widx(ci0, si0)
        pltpu.sync_copy(ones_v, deg_sh.at[ci0], add=True)
        idxc(g + 2, ci0, si0)
        widx(ci1, si1)
        pltpu.sync_copy(ones_v, deg_sh.at[ci1], add=True)
        idxc(g + 3, ci1, si1)
        return 0

    lax.fori_loop(0, DEG_CPT // 2 - 1, body, 0)
    widx(ci0, si0)
    pltpu.sync_copy(ones_v, deg_sh.at[ci0], add=True)
    widx(ci1, si1)
    pltpu.sync_copy(ones_v, deg_sh.at[ci1], add=True)
    plsc.subcore_barrier()

    pltpu.sync_copy(
        deg_sh.at[pl.ds(sid * ROWS_PER_TILE_PAD, ROWS_PER_TILE_PAD)],
        deg_hbm.at[cid, pl.ds(sid * ROWS_PER_TILE_PAD, ROWS_PER_TILE_PAD)],
    )


NB = 4
NI = 8
GD = 2
SD = 2
ID = 6
UNROLL = 8


@functools.partial(
    pl.kernel,
    out_type=jax.ShapeDtypeStruct((NC, NPAD, D), jnp.float32),
    mesh=_mesh,
    scratch_types=[
        pltpu.VMEM_SHARED((NPAD, D), jnp.float32),
        [pltpu.VMEM((CHUNK,), jnp.int32) for _ in range(NI)],
        [pltpu.VMEM((CHUNK,), jnp.int32) for _ in range(NI)],
        [pltpu.VMEM((CHUNK, D), jnp.float32) for _ in range(NB)],
        [pltpu.SemaphoreType.DMA for _ in range(NI)],
        [pltpu.SemaphoreType.DMA for _ in range(NB)],
        [pltpu.SemaphoreType.DMA for _ in range(NB)],
    ],
)
def _sc_segsum(z_hbm, row_hbm, col_hbm, out_hbm, acc_sh, RI, CI, GB, SI, SG, SS):
    cid = lax.axis_index("c")
    sid = lax.axis_index("s")
    w = cid * NS + sid

    z16 = jnp.zeros((16,), jnp.float32)

    def zbody(r, _):
        for cc in range(D // 16):
            GB[NB - 1][r, pl.ds(cc * 16, 16)] = z16
        return 0

    lax.fori_loop(0, CHUNK, zbody, 0)
    for jj in range(ROWS_PER_TILE_PAD // CHUNK):
        pltpu.async_copy(
            GB[NB - 1],
            acc_sh.at[pl.ds(sid * ROWS_PER_TILE_PAD + jj * CHUNK, CHUNK)],
            SS[NB - 1],
        )

    def idxc(c, m):
        base = pl.multiple_of(w * EDGES_PER_TILE + c * CHUNK, CHUNK)
        pltpu.async_copy(row_hbm.at[pl.ds(base, CHUNK)], RI[m], SI[m])
        pltpu.async_copy(col_hbm.at[pl.ds(base, CHUNK)], CI[m], SI[m])

    def widx(m):
        pltpu.make_async_copy(row_hbm.at[pl.ds(0, CHUNK)], RI[m], SI[m]).wait()
        pltpu.make_async_copy(col_hbm.at[pl.ds(0, CHUNK)], CI[m], SI[m]).wait()

    def gath(m, j):
        pltpu.async_copy(z_hbm.at[RI[m]], GB[j], SG[j])

    def wgath(j):
        pltpu.make_async_copy(z_hbm.at[pl.ds(0, CHUNK)], GB[j], SG[j]).wait()

    def scat(j, m):
        pltpu.async_copy(GB[j], acc_sh.at[CI[m]], SS[j], add=True)

    def wscat(j):
        pltpu.make_async_copy(GB[j], acc_sh.at[pl.ds(0, CHUNK)], SS[j]).wait()

    def step(t, u, do_wscat=True, do_gather=True, do_idx=True):
        if do_wscat:
            wscat((u + NB - SD) % NB)
        if do_gather:
            widx((u + GD) % NI)
            gath((u + GD) % NI, (u + GD) % NB)
        wgath(u % NB)
        scat(u % NB, u % NI)
        if do_idx:
            idxc(t + ID, (u + ID) % NI)

    for m in range(ID):
        idxc(m, m)
    for c in range(GD):
        widx(c % NI)
        gath(c % NI, c % NB)
    for jj in range(ROWS_PER_TILE_PAD // CHUNK):
        wscat(NB - 1)
    plsc.subcore_barrier()
    for t in range(UNROLL):
        step(t, t, do_wscat=(t >= SD))

    def body(k, _):
        t0 = k * UNROLL
        for u in range(UNROLL):
            step(t0 + u, u)
        return 0

    lax.fori_loop(1, CH_PER_TILE // UNROLL - 1, body, 0)

    for t in range(CH_PER_TILE - UNROLL, CH_PER_TILE):
        step(
            t,
            t % UNROLL,
            do_gather=(t + GD < CH_PER_TILE),
            do_idx=(t + ID < CH_PER_TILE),
        )
    for t in range(CH_PER_TILE - SD, CH_PER_TILE):
        wscat(t % NB)
    plsc.subcore_barrier()

    pltpu.sync_copy(
        acc_sh.at[pl.ds(sid * ROWS_PER_TILE_PAD, ROWS_PER_TILE_PAD)],
        out_hbm.at[cid, pl.ds(sid * ROWS_PER_TILE_PAD, ROWS_PER_TILE_PAD)],
    )


def _tc_first_body(x_ref, w_ref, d0_ref, d1_ref, z_ref, dinv_ref):
    dinv = lax.rsqrt(d0_ref[:N] + d1_ref[:N] + 1.0)
    dinv_ref[...] = dinv
    hw = jnp.dot(x_ref[...], w_ref[...], preferred_element_type=jnp.float32)
    z_ref[...] = dinv * hw


def _bn_gelu(p_ref, z_ref, dinv_ref, b_ref, g_ref, be_ref):
    conv = dinv_ref[...] * (p_ref[0, :N] + p_ref[1, :N] + z_ref[...]) + b_ref[...]
    mu = jnp.mean(conv, axis=0, keepdims=True)
    var = jnp.mean((conv - mu) ** 2, axis=0, keepdims=True)
    hn = (conv - mu) * lax.rsqrt(var + EPS) * g_ref[...] + be_ref[...]
    return 0.5 * hn * (1.0 + lax.erf(hn * 0.7071067811865476))


def _tc_mid_body(p_ref, z_ref, dinv_ref, b_ref, g_ref, be_ref, wn_ref, zn_ref):
    act = _bn_gelu(p_ref, z_ref, dinv_ref, b_ref, g_ref, be_ref)
    zn_ref[...] = dinv_ref[...] * jnp.dot(
        act, wn_ref[...], preferred_element_type=jnp.float32
    )


def _tc_last_body(p_ref, z_ref, dinv_ref, b_ref, g_ref, be_ref, aw_ref, out_ref):
    act = _bn_gelu(p_ref, z_ref, dinv_ref, b_ref, g_ref, be_ref)
    out_ref[...] = act * aw_ref[...]


_tc_first = pl.pallas_call(
    _tc_first_body,
    out_shape=(
        jax.ShapeDtypeStruct((N, D), jnp.float32),
        jax.ShapeDtypeStruct((N, 1), jnp.float32),
    ),
)
_tc_mid = pl.pallas_call(
    _tc_mid_body, out_shape=jax.ShapeDtypeStruct((N, D), jnp.float32)
)
_tc_last = pl.pallas_call(
    _tc_last_body, out_shape=jax.ShapeDtypeStruct((N, D), jnp.float32)
)


def kernel(x, edge_index, W, b, gamma, beta, attn_w):
    row = edge_index[0].astype(jnp.int32)
    col = edge_index[1].astype(jnp.int32)

    npad_e = E_PAD - E
    j = jnp.arange(npad_e, dtype=jnp.int32)
    row_p = jnp.concatenate([row, j % N])
    col_p = jnp.concatenate([col, N + (j % (NPAD - N))])

    deg_p = _sc_deg(col_p)
    d0 = deg_p[0].reshape(NPAD, 1)
    d1 = deg_p[1].reshape(NPAD, 1)

    b2 = b.reshape(L, 1, D)
    g2 = gamma.reshape(L, 1, D)
    be2 = beta.reshape(L, 1, D)
    aw2 = attn_w.reshape(1, D)

    z, dinv2d = _tc_first(x, W[0], d0, d1)
    for i in range(L - 1):
        p = _sc_segsum(z, row_p, col_p)
        z = _tc_mid(p, z, dinv2d, b2[i], g2[i], be2[i], W[i + 1])
    p = _sc_segsum(z, row_p, col_p)
    return _tc_last(p, z, dinv2d, b2[L - 1], g2[L - 1], be2[L - 1], aw2)


# speedup vs baseline: 1.0964x; 1.0964x over previous
import functools

import jax
import jax.numpy as jnp
from jax import lax
from jax.experimental import pallas as pl
from jax.experimental.pallas import tpu as pltpu
from jax.experimental.pallas import tpu_sc as plsc

N = 10000
E = 320000
D = 128
L = 4
EPS = 1e-5

NC = 2
NS = 16
NW = NC * NS

CHUNK = 64
CH_PER_TILE = 160
EDGES_PER_TILE = CH_PER_TILE * CHUNK
E_PAD = NW * EDGES_PER_TILE
DEG_CHUNK = 128
DEG_CPT = EDGES_PER_TILE // DEG_CHUNK
NPAD = 10240
ROWS_PER_TILE_PAD = NPAD // NS

_mesh = plsc.VectorSubcoreMesh(
    core_axis_name="c", subcore_axis_name="s", num_cores=NC, num_subcores=NS
)


@functools.partial(
    pl.kernel,
    out_type=jax.ShapeDtypeStruct((NC, NPAD), jnp.float32),
    mesh=_mesh,
    scratch_types=[
        pltpu.VMEM_SHARED((NPAD,), jnp.float32),
        pltpu.VMEM((DEG_CHUNK,), jnp.int32),
        pltpu.VMEM((DEG_CHUNK,), jnp.int32),
        pltpu.VMEM((DEG_CHUNK,), jnp.float32),
        pltpu.VMEM((ROWS_PER_TILE_PAD,), jnp.float32),
        pltpu.SemaphoreType.DMA,
        pltpu.SemaphoreType.DMA,
    ],
)
def _sc_deg(col_hbm, deg_hbm, deg_sh, ci0, ci1, ones_v, dbuf, si0, si1):
    cid = lax.axis_index("c")
    sid = lax.axis_index("s")
    w = cid * NS + sid

    z16 = jnp.zeros((16,), jnp.float32)
    o16 = jnp.ones((16,), jnp.float32)
    for i in range(DEG_CHUNK // 16):
        ones_v[pl.ds(i * 16, 16)] = o16

    def zbody(i, _):
        dbuf[pl.ds(i * 16, 16)] = z16
        return 0

    lax.fori_loop(0, ROWS_PER_TILE_PAD // 16, zbody, 0)
    pltpu.sync_copy(dbuf, deg_sh.at[pl.ds(sid * ROWS_PER_TILE_PAD, ROWS_PER_TILE_PAD)])
    plsc.subcore_barrier()

    def idxc(g, ci, si):
        base = pl.multiple_of(w * EDGES_PER_TILE + g * DEG_CHUNK, DEG_CHUNK)
        pltpu.async_copy(col_hbm.at[pl.ds(base, DEG_CHUNK)], ci, si)

    def widx(ci, si):
        pltpu.make_async_copy(col_hbm.at[pl.ds(0, DEG_CHUNK)], ci, si).wait()

    idxc(0, ci0, si0)
    idxc(1, ci1, si1)

    def body(k, _):
        g = k * 2
        widx(ci0, si0)
        pltpu.sync_copy(ones_v, deg_sh.at[ci0], add=True)
        idxc(g + 2, ci0, si0)
        widx(ci1, si1)
        pltpu.sync_copy(ones_v, deg_sh.at[ci1], add=True)
        idxc(g + 3, ci1, si1)
        return 0

    lax.fori_loop(0, DEG_CPT // 2 - 1, body, 0)
    widx(ci0, si0)
    pltpu.sync_copy(ones_v, deg_sh.at[ci0], add=True)
    widx(ci1, si1)
    pltpu.sync_copy(ones_v, deg_sh.at[ci1], add=True)
    plsc.subcore_barrier()

    pltpu.sync_copy(
        deg_sh.at[pl.ds(sid * ROWS_PER_TILE_PAD, ROWS_PER_TILE_PAD)],
        deg_hbm.at[cid, pl.ds(sid * ROWS_PER_TILE_PAD, ROWS_PER_TILE_PAD)],
    )


NB = 4
NI = 8
GD = 3
SD = 1
ID = 6
UNROLL = 8


@functools.partial(
    pl.kernel,
    out_type=jax.ShapeDtypeStruct((NC, NPAD, D), jnp.float32),
    mesh=_mesh,
    scratch_types=[
        pltpu.VMEM_SHARED((NPAD, D), jnp.float32),
        [pltpu.VMEM((CHUNK,), jnp.int32) for _ in range(NI)],
        [pltpu.VMEM((CHUNK,), jnp.int32) for _ in range(NI)],
        [pltpu.VMEM((CHUNK, D), jnp.float32) for _ in range(NB)],
        [pltpu.SemaphoreType.DMA for _ in range(NI)],
        [pltpu.SemaphoreType.DMA for _ in range(NB)],
        [pltpu.SemaphoreType.DMA for _ in range(NB)],
    ],
)
def _sc_segsum(z_hbm, row_hbm, col_hbm, out_hbm, acc_sh, RI, CI, GB, SI, SG, SS):
    cid = lax.axis_index("c")
    sid = lax.axis_index("s")
    w = cid * NS + sid

    z16 = jnp.zeros((16,), jnp.float32)

    def zbody(r, _):
        for cc in range(D // 16):
            GB[NB - 1][r, pl.ds(cc * 16, 16)] = z16
        return 0

    lax.fori_loop(0, CHUNK, zbody, 0)
    for jj in range(ROWS_PER_TILE_PAD // CHUNK):
        pltpu.async_copy(
            GB[NB - 1],
            acc_sh.at[pl.ds(sid * ROWS_PER_TILE_PAD + jj * CHUNK, CHUNK)],
            SS[NB - 1],
        )

    def idxc(c, m):
        base = pl.multiple_of(w * EDGES_PER_TILE + c * CHUNK, CHUNK)
        pltpu.async_copy(row_hbm.at[pl.ds(base, CHUNK)], RI[m], SI[m])
        pltpu.async_copy(col_hbm.at[pl.ds(base, CHUNK)], CI[m], SI[m])

    def widx(m):
        pltpu.make_async_copy(row_hbm.at[pl.ds(0, CHUNK)], RI[m], SI[m]).wait()
        pltpu.make_async_copy(col_hbm.at[pl.ds(0, CHUNK)], CI[m], SI[m]).wait()

    def gath(m, j):
        pltpu.async_copy(z_hbm.at[RI[m]], GB[j], SG[j])

    def wgath(j):
        pltpu.make_async_copy(z_hbm.at[pl.ds(0, CHUNK)], GB[j], SG[j]).wait()

    def scat(j, m):
        pltpu.async_copy(GB[j], acc_sh.at[CI[m]], SS[j], add=True)

    def wscat(j):
        pltpu.make_async_copy(GB[j], acc_sh.at[pl.ds(0, CHUNK)], SS[j]).wait()

    def step(t, u, do_wscat=True, do_gather=True, do_idx=True):
        if do_wscat:
            wscat((u + NB - SD) % NB)
        if do_gather:
            widx((u + GD) % NI)
            gath((u + GD) % NI, (u + GD) % NB)
        wgath(u % NB)
        scat(u % NB, u % NI)
        if do_idx:
            idxc(t + ID, (u + ID) % NI)

    for m in range(ID):
        idxc(m, m)
    for c in range(GD):
        widx(c % NI)
        gath(c % NI, c % NB)
    for jj in range(ROWS_PER_TILE_PAD // CHUNK):
        wscat(NB - 1)
    plsc.subcore_barrier()
    for t in range(UNROLL):
        step(t, t, do_wscat=(t >= SD))

    def body(k, _):
        t0 = k * UNROLL
        for u in range(UNROLL):
            step(t0 + u, u)
        return 0

    lax.fori_loop(1, CH_PER_TILE // UNROLL - 1, body, 0)

    for t in range(CH_PER_TILE - UNROLL, CH_PER_TILE):
        step(
            t,
            t % UNROLL,
            do_gather=(t + GD < CH_PER_TILE),
            do_idx=(t + ID < CH_PER_TILE),
        )
    for t in range(CH_PER_TILE - SD, CH_PER_TILE):
        wscat(t % NB)
    plsc.subcore_barrier()

    pltpu.sync_copy(
        acc_sh.at[pl.ds(sid * ROWS_PER_TILE_PAD, ROWS_PER_TILE_PAD)],
        out_hbm.at[cid, pl.ds(sid * ROWS_PER_TILE_PAD, ROWS_PER_TILE_PAD)],
    )


def _tc_first_body(x_ref, w_ref, d0_ref, d1_ref, z_ref, dinv_ref):
    dinv = lax.rsqrt(d0_ref[:N] + d1_ref[:N] + 1.0)
    dinv_ref[...] = dinv
    hw = jnp.dot(x_ref[...], w_ref[...], preferred_element_type=jnp.float32)
    z_ref[...] = dinv * hw


def _bn_gelu(p_ref, z_ref, dinv_ref, b_ref, g_ref, be_ref):
    conv = dinv_ref[...] * (p_ref[0, :N] + p_ref[1, :N] + z_ref[...]) + b_ref[...]
    mu = jnp.mean(conv, axis=0, keepdims=True)
    var = jnp.mean((conv - mu) ** 2, axis=0, keepdims=True)
    hn = (conv - mu) * lax.rsqrt(var + EPS) * g_ref[...] + be_ref[...]
    return 0.5 * hn * (1.0 + lax.erf(hn * 0.7071067811865476))


def _tc_mid_body(p_ref, z_ref, dinv_ref, b_ref, g_ref, be_ref, wn_ref, zn_ref):
    act = _bn_gelu(p_ref, z_ref, dinv_ref, b_ref, g_ref, be_ref)
    zn_ref[...] = dinv_ref[...] * jnp.dot(
        act, wn_ref[...], preferred_element_type=jnp.float32
    )


def _tc_last_body(p_ref, z_ref, dinv_ref, b_ref, g_ref, be_ref, aw_ref, out_ref):
    act = _bn_gelu(p_ref, z_ref, dinv_ref, b_ref, g_ref, be_ref)
    out_ref[...] = act * aw_ref[...]


_tc_first = pl.pallas_call(
    _tc_first_body,
    out_shape=(
        jax.ShapeDtypeStruct((N, D), jnp.float32),
        jax.ShapeDtypeStruct((N, 1), jnp.float32),
    ),
)
_tc_mid = pl.pallas_call(
    _tc_mid_body, out_shape=jax.ShapeDtypeStruct((N, D), jnp.float32)
)
_tc_last = pl.pallas_call(
    _tc_last_body, out_shape=jax.ShapeDtypeStruct((N, D), jnp.float32)
)


def kernel(x, edge_index, W, b, gamma, beta, attn_w):
    row = edge_index[0].astype(jnp.int32)
    col = edge_index[1].astype(jnp.int32)

    npad_e = E_PAD - E
    j = jnp.arange(npad_e, dtype=jnp.int32)
    row_p = jnp.concatenate([row, j % N])
    col_p = jnp.concatenate([col, N + (j % (NPAD - N))])

    deg_p = _sc_deg(col_p)
    d0 = deg_p[0].reshape(NPAD, 1)
    d1 = deg_p[1].reshape(NPAD, 1)

    b2 = b.reshape(L, 1, D)
    g2 = gamma.reshape(L, 1, D)
    be2 = beta.reshape(L, 1, D)
    aw2 = attn_w.reshape(1, D)

    z, dinv2d = _tc_first(x, W[0], d0, d1)
    for i in range(L - 1):
        p = _sc_segsum(z, row_p, col_p)
        z = _tc_mid(p, z, dinv2d, b2[i], g2[i], be2[i], W[i + 1])
    p = _sc_segsum(z, row_p, col_p)
    return _tc_last(p, z, dinv2d, b2[L - 1], g2[L - 1], be2[L - 1], aw2)
